# baseline (speedup 1.0000x reference)
import jax
import jax.numpy as jnp
from jax import lax
from jax.experimental import pallas as pl
from jax.experimental.pallas import tpu as pltpu

_B, _S, _C = 4, 8192, 768
_K = 8
_G = _S // _K
_SG = _G // _K
_NCAND = _K * _K

_CH = 2048
_NCH = _S // _CH
_GPC = _CH // _K

_NEG_INF = float("-inf")
_I32_MAX = 2**31 - 1


def _body(x_ref, out_ref, l1_ref, gsel_ref, cand_ref):
    p = pl.program_id(1)
    j = pl.program_id(2)

    x3 = x_ref[0].reshape(_GPC, _K, _C)

    @pl.when(p == 0)
    def _sweep1():
        l1_ref[pl.ds(j * _GPC, _GPC), :] = jnp.max(x3, axis=1)

        @pl.when(j == _NCH - 1)
        def _select():
            l1v = l1_ref[...].reshape(_SG, _K, _C)
            l2 = jnp.max(l1v, axis=1)

            sg_iota = lax.broadcasted_iota(jnp.int32, (_SG, _C), 0)
            sels = []
            for r in range(_K):
                m = jnp.max(l2, axis=0, keepdims=True)
                sel = jnp.max(jnp.where(l2 == m, sg_iota, -1), axis=0,
                              keepdims=True)
                sels.append(sel)
                if r < _K - 1:
                    l2 = jnp.where(sg_iota == sel, _NEG_INF, l2)

            sg_iota3 = lax.broadcasted_iota(jnp.int32, (_SG, 1, _C), 0)
            row8 = lax.broadcasted_iota(jnp.int32, (_K, _C), 0)
            cand_v = []
            cand_g = []
            for r in range(_K):
                eq = sg_iota3 == sels[r].reshape(1, 1, _C)
                cand_v.append(jnp.max(jnp.where(eq, l1v, _NEG_INF), axis=0))
                cand_g.append(sels[r] * _K + row8)
            cv = jnp.concatenate(cand_v, axis=0)
            cg = jnp.concatenate(cand_g, axis=0)

            gs = []
            for r in range(_K):
                m = jnp.max(cv, axis=0, keepdims=True)
                g = jnp.max(jnp.where(cv == m, cg, -1), axis=0,
                            keepdims=True)
                gs.append(g)
                if r < _K - 1:
                    cv = jnp.where(cg == g, _NEG_INF, cv)
            gsel_ref[...] = jnp.concatenate(gs, axis=0)

    @pl.when(p == 1)
    def _sweep2():
        gsel = gsel_ref[...]
        gi3 = lax.broadcasted_iota(jnp.int32, (_GPC, 1, _C), 0) + j * _GPC
        contribs = []
        for r in range(_K):
            eq = gi3 == gsel[r].reshape(1, 1, _C)
            contribs.append(jnp.max(jnp.where(eq, x3, _NEG_INF), axis=0))
        contrib = jnp.concatenate(contribs, axis=0)

        @pl.when(j == 0)
        def _():
            cand_ref[...] = contrib

        @pl.when(j > 0)
        def _():
            cand_ref[...] = jnp.maximum(cand_ref[...], contrib)

        @pl.when(j == _NCH - 1)
        def _finalize():
            cv = cand_ref[...]
            row8 = lax.broadcasted_iota(jnp.int32, (_K, _C), 0)
            s64 = jnp.concatenate(
                [gsel[r].reshape(1, _C) * _K + row8 for r in range(_K)],
                axis=0)

            kept_v = []
            kept_s = []
            for r in range(_K):
                m = jnp.max(cv, axis=0, keepdims=True)
                ps = jnp.max(jnp.where(cv == m, s64, -1), axis=0,
                             keepdims=True)
                kept_v.append(m)
                kept_s.append(ps)
                if r < _K - 1:
                    cv = jnp.where(s64 == ps, _NEG_INF, cv)
            av = jnp.concatenate(kept_v, axis=0)
            ai = jnp.concatenate(kept_s, axis=0)

            outs = []
            for r in range(_K):
                mi = jnp.min(ai, axis=0, keepdims=True)
                outs.append(jnp.max(jnp.where(ai == mi, av, _NEG_INF),
                                    axis=0, keepdims=True))
                if r < _K - 1:
                    ai = jnp.where(ai == mi, _I32_MAX, ai)
            out_ref[0] = jnp.concatenate(outs, axis=0)


def kernel(x):
    return pl.pallas_call(
        _body,
        grid=(_B, 2, _NCH),
        in_specs=[pl.BlockSpec((1, _CH, _C), lambda b, p, j: (b, j, 0))],
        out_specs=pl.BlockSpec((1, _K, _C), lambda b, p, j: (b, 0, 0)),
        out_shape=jax.ShapeDtypeStruct((_B, _K, _C), jnp.float32),
        scratch_shapes=[
            pltpu.VMEM((_G, _C), jnp.float32),
            pltpu.VMEM((_K, _C), jnp.int32),
            pltpu.VMEM((_NCAND, _C), jnp.float32),
        ],
    )(x)


# speedup vs baseline: 1.2902x; 1.2902x over previous
import jax
import jax.numpy as jnp
from jax import lax
from jax.experimental import pallas as pl
from jax.experimental.pallas import tpu as pltpu

_B, _S, _C = 4, 8192, 768
_K = 8

_CH = 2048
_NCH = _S // _CH
_GPC = _CH // _K
_NCAND = _K * _K
_TCAND = _NCH * _NCAND

_NEG_INF = float("-inf")
_I32_MAX = 2**31 - 1


def _body(x_ref, out_ref, cv_ref, ci_ref):
    j = pl.program_id(1)

    x2 = x_ref[0]
    x3 = x2.reshape(_GPC, _K, _C)
    l1c = jnp.max(x3, axis=1)

    g_iota = lax.broadcasted_iota(jnp.int32, (_GPC, _C), 0)
    sels = []
    for r in range(_K):
        m = jnp.max(l1c, axis=0, keepdims=True)
        sel = jnp.max(jnp.where(l1c == m, g_iota, -1), axis=0,
                      keepdims=True)
        sels.append(sel)
        if r < _K - 1:
            l1c = jnp.where(g_iota == sel, _NEG_INF, l1c)

    grp2 = lax.broadcasted_iota(jnp.int32, (_CH, _C), 0) // _K
    row8 = lax.broadcasted_iota(jnp.int32, (_K, _C), 0)
    vals = []
    idxs = []
    for r in range(_K):
        masked = jnp.where(grp2 == sels[r], x2, _NEG_INF)
        vals.append(jnp.max(masked.reshape(_GPC, _K, _C), axis=0))
        idxs.append((j * _GPC + sels[r]) * _K + row8)
    cv_ref[pl.ds(j * _NCAND, _NCAND), :] = jnp.concatenate(vals, axis=0)
    ci_ref[pl.ds(j * _NCAND, _NCAND), :] = jnp.concatenate(idxs, axis=0)

    @pl.when(j == _NCH - 1)
    def _finalize():
        cv = cv_ref[...]
        ci = ci_ref[...]

        kept_v = []
        kept_s = []
        for r in range(_K):
            m = jnp.max(cv, axis=0, keepdims=True)
            ps = jnp.max(jnp.where(cv == m, ci, -1), axis=0, keepdims=True)
            kept_v.append(m)
            kept_s.append(ps)
            if r < _K - 1:
                cv = jnp.where(ci == ps, _NEG_INF, cv)
        av = jnp.concatenate(kept_v, axis=0)
        ai = jnp.concatenate(kept_s, axis=0)

        outs = []
        for r in range(_K):
            mi = jnp.min(ai, axis=0, keepdims=True)
            outs.append(jnp.max(jnp.where(ai == mi, av, _NEG_INF), axis=0,
                                keepdims=True))
            if r < _K - 1:
                ai = jnp.where(ai == mi, _I32_MAX, ai)
        out_ref[0] = jnp.concatenate(outs, axis=0)


def kernel(x):
    return pl.pallas_call(
        _body,
        grid=(_B, _NCH),
        in_specs=[pl.BlockSpec((1, _CH, _C), lambda b, j: (b, j, 0))],
        out_specs=pl.BlockSpec((1, _K, _C), lambda b, j: (b, 0, 0)),
        out_shape=jax.ShapeDtypeStruct((_B, _K, _C), jnp.float32),
        scratch_shapes=[
            pltpu.VMEM((_TCAND, _C), jnp.float32),
            pltpu.VMEM((_TCAND, _C), jnp.int32),
        ],
    )(x)
